# baseline (speedup 1.0000x reference)
import functools

import jax
import jax.numpy as jnp
from jax.experimental import pallas as pl


def _chamfer_body(pred_ref, targett_ref, out_ref, *, n, m, block_n):
    tt = targett_ref[...]
    tn = tt[0:1, :] * tt[0:1, :] + tt[1:2, :] * tt[1:2, :] \
        + tt[2:3, :] * tt[2:3, :]

    n_chunks = 4
    mc = m // n_chunks

    def body(i, carry):
        col_min, row_sum = carry
        p = pred_ref[pl.ds(i * block_n, block_n), :]
        pn = jnp.sum(p * p, axis=1, keepdims=True)
        pblk = -2.0 * p
        row_min = None
        col_parts = []
        for c in range(n_chunks):
            cross = jax.lax.dot_general(
                pblk, tt[:, c * mc:(c + 1) * mc],
                (((1,), (0,)), ((), ())),
                preferred_element_type=jnp.float32)
            d2 = (pn + tn[:, c * mc:(c + 1) * mc]) + cross
            rm = jnp.min(d2, axis=1, keepdims=True)
            row_min = rm if row_min is None else jnp.minimum(row_min, rm)
            col_parts.append(jnp.min(d2, axis=0, keepdims=True))
        row_sum = row_sum + jnp.sum(
            jnp.sqrt(jnp.maximum(row_min, 0.0) + 1e-12))
        col_min = jnp.minimum(col_min, jnp.concatenate(col_parts, axis=1))
        return col_min, row_sum

    col_min, row_sum = jax.lax.fori_loop(
        0, n // block_n, body,
        (jnp.full((1, m), jnp.inf, dtype=jnp.float32),
         jnp.zeros((1, 1), dtype=jnp.float32)))
    back = jnp.sum(jnp.sqrt(jnp.maximum(col_min, 0.0) + 1e-12),
                   axis=1, keepdims=True)
    out_ref[...] = (row_sum / n + back / m) * 0.5


def kernel(pred, target):
    pred = pred.astype(jnp.float32)
    target = target.astype(jnp.float32)
    n, k = pred.shape
    m, _ = target.shape
    pred_rows = jnp.pad(pred, ((0, 0), (0, 8 - k)))
    targett = jnp.pad(target.T, ((0, 8 - k), (0, 0)))
    out = pl.pallas_call(
        functools.partial(_chamfer_body, n=n, m=m, block_n=512),
        out_shape=jax.ShapeDtypeStruct((1, 1), jnp.float32),
    )(pred_rows, targett)
    return out[0, 0]


# speedup vs baseline: 1.0480x; 1.0480x over previous
import functools

import jax
import jax.numpy as jnp
from jax.experimental import pallas as pl


def _chamfer_body(pred_ref, targett_ref, out_ref, *, n, m, block_n):
    tt = targett_ref[...]
    tn = tt[0:1, :] * tt[0:1, :] + tt[1:2, :] * tt[1:2, :] \
        + tt[2:3, :] * tt[2:3, :]

    n_chunks = 8
    mc = m // n_chunks

    def body(i, carry):
        col_min, row_sum = carry
        p = pred_ref[pl.ds(i * block_n, block_n), :]
        pn = jnp.sum(p * p, axis=1, keepdims=True)
        pblk = -2.0 * p
        row_min = None
        col_parts = []
        for c in range(n_chunks):
            cross = jax.lax.dot_general(
                pblk, tt[:, c * mc:(c + 1) * mc],
                (((1,), (0,)), ((), ())),
                preferred_element_type=jnp.float32)
            d2 = (pn + tn[:, c * mc:(c + 1) * mc]) + cross
            rm = jnp.min(d2, axis=1, keepdims=True)
            row_min = rm if row_min is None else jnp.minimum(row_min, rm)
            col_parts.append(jnp.min(d2, axis=0, keepdims=True))
        row_sum = row_sum + jnp.sum(
            jnp.sqrt(jnp.maximum(row_min, 0.0) + 1e-12))
        col_min = jnp.minimum(col_min, jnp.concatenate(col_parts, axis=1))
        return col_min, row_sum

    col_min, row_sum = jax.lax.fori_loop(
        0, n // block_n, body,
        (jnp.full((1, m), jnp.inf, dtype=jnp.float32),
         jnp.zeros((1, 1), dtype=jnp.float32)))
    back = jnp.sum(jnp.sqrt(jnp.maximum(col_min, 0.0) + 1e-12),
                   axis=1, keepdims=True)
    out_ref[...] = (row_sum / n + back / m) * 0.5


def kernel(pred, target):
    pred = pred.astype(jnp.float32)
    target = target.astype(jnp.float32)
    n, k = pred.shape
    m, _ = target.shape
    pred_rows = jnp.pad(pred, ((0, 0), (0, 8 - k)))
    targett = jnp.pad(target.T, ((0, 8 - k), (0, 0)))
    out = pl.pallas_call(
        functools.partial(_chamfer_body, n=n, m=m, block_n=1024),
        out_shape=jax.ShapeDtypeStruct((1, 1), jnp.float32),
    )(pred_rows, targett)
    return out[0, 0]
